# baseline (speedup 1.0000x reference)
import jax
import jax.numpy as jnp
from jax import lax
from jax.experimental import pallas as pl
from jax.experimental.pallas import tpu as pltpu
from jax.experimental.pallas import tpu_sc as plsc

SEQ = 200
BATCH = 4096
EMBED = 64
VOCAB = 1000000
PHYS = VOCAB // 2
PW = 2 * EMBED
NC = 2
NS = 16
NW = NC * NS
BPW = BATCH // NW
LANES = 16
NITER = SEQ // 4 - 1


NBLK = 3906
KMAIN = 122


def _conv_body(embt_hbm, tab_hbm, in0, in1, ot0, ot1, tin, tout,
               sem_i, sem_o):
    wid = lax.axis_index("s") * NC + lax.axis_index("c")
    iota = lax.iota(jnp.int32, LANES)

    def fire_in(buf, k):
        g = k * NW + wid
        pltpu.async_copy(embt_hbm.at[:, pl.ds(g * 256, 256)], buf, sem_i)

    def wait_in(buf, k):
        g = k * NW + wid
        pltpu.make_async_copy(
            embt_hbm.at[:, pl.ds(g * 256, 256)], buf, sem_i).wait()

    def fire_out(buf, k):
        g = k * NW + wid
        pltpu.async_copy(buf, tab_hbm.at[pl.ds(g * 128, 128)], sem_o)

    def wait_out(buf, k):
        g = k * NW + wid
        pltpu.make_async_copy(
            buf, tab_hbm.at[pl.ds(g * 128, 128)], sem_o).wait()

    def transpose_block(inb, outb):
        @pl.loop(0, 128)
        def _(p):
            for h in range(2):
                col = 2 * p + h + jnp.zeros((LANES,), jnp.int32)
                for dc in range(4):
                    v = plsc.load_gather(inb, [dc * LANES + iota, col])
                    outb[p, pl.ds(h * EMBED + dc * LANES, LANES)] = v

    fire_in(in0, 0)
    fire_in(in1, 1)

    @pl.loop(0, KMAIN // 2 - 1)
    def _(j):
        k0 = 2 * j
        wait_in(in0, k0)

        @pl.when(j > 0)
        def _():
            wait_out(ot0, k0 - 2)

        transpose_block(in0, ot0)
        fire_out(ot0, k0)
        fire_in(in0, k0 + 2)
        wait_in(in1, k0 + 1)

        @pl.when(j > 0)
        def _():
            wait_out(ot1, k0 - 1)

        transpose_block(in1, ot1)
        fire_out(ot1, k0 + 1)
        fire_in(in1, k0 + 3)

    wait_in(in0, KMAIN - 2)
    wait_out(ot0, KMAIN - 4)
    transpose_block(in0, ot0)
    fire_out(ot0, KMAIN - 2)
    wait_in(in1, KMAIN - 1)
    wait_out(ot1, KMAIN - 3)
    transpose_block(in1, ot1)
    fire_out(ot1, KMAIN - 1)

    @pl.when(wid < 2)
    def _():
        fire_in(in0, KMAIN)
        wait_in(in0, KMAIN)
        wait_out(ot0, KMAIN - 2)
        transpose_block(in0, ot0)
        fire_out(ot0, KMAIN)

    @pl.when(wid == 2)
    def _():
        pltpu.sync_copy(embt_hbm.at[:, pl.ds(NBLK * 256, 64)], tin)

        @pl.loop(0, 32)
        def _(p):
            for h in range(2):
                col = 2 * p + h + jnp.zeros((LANES,), jnp.int32)
                for dc in range(4):
                    v = plsc.load_gather(tin, [dc * LANES + iota, col])
                    tout[p, pl.ds(h * EMBED + dc * LANES, LANES)] = v

        pltpu.sync_copy(tout, tab_hbm.at[pl.ds(NBLK * 128, 32)])

    @pl.when(wid < 2)
    def _():
        wait_out(ot0, KMAIN)

    @pl.when(wid >= 2)
    def _():
        wait_out(ot0, KMAIN - 2)

    wait_out(ot1, KMAIN - 1)


_sc_convert = pl.kernel(
    _conv_body,
    out_type=jax.ShapeDtypeStruct((PHYS, PW), jnp.float32),
    mesh=plsc.VectorSubcoreMesh(core_axis_name="c", subcore_axis_name="s"),
    scratch_types=[
        pltpu.VMEM((EMBED, 256), jnp.float32),
        pltpu.VMEM((EMBED, 256), jnp.float32),
        pltpu.VMEM((128, PW), jnp.float32),
        pltpu.VMEM((128, PW), jnp.float32),
        pltpu.VMEM((EMBED, 64), jnp.float32),
        pltpu.VMEM((32, PW), jnp.float32),
        pltpu.SemaphoreType.DMA,
        pltpu.SemaphoreType.DMA,
    ],
    compiler_params=pltpu.CompilerParams(
        needs_layout_passes=False, use_tc_tiling_on_sc=True),
)


def _body(idx_hbm, tab_hbm, wb_hbm, out_hbm,
          idx_v, idx2_v, acc_v, a0, a1, b0, b1, out_v, wb_v,
          sem_a, sem_b):
    wid = lax.axis_index("s") * NC + lax.axis_index("c")

    pltpu.sync_copy(wb_hbm, wb_v)
    pltpu.sync_copy(idx_hbm.at[:, pl.ds(wid * BPW, BPW)], idx_v)

    zeros = jnp.zeros((LANES,), jnp.float32)
    one = jnp.int32(1)
    iota = lax.iota(jnp.int32, LANES)

    @pl.loop(0, SEQ)
    def _(s):
        for c in range(BPW // LANES):
            sl = pl.ds(c * LANES, LANES)
            idx2_v[s, sl] = lax.shift_right_logical(idx_v[s, sl], one)

    @pl.loop(0, EMBED)
    def _(d):
        for c in range(BPW // LANES):
            acc_v[d, pl.ds(c * LANES, LANES)] = zeros

    def fire(bufs, sem, s):
        pltpu.async_copy(tab_hbm.at[idx2_v.at[s]], bufs[0], sem)
        pltpu.async_copy(tab_hbm.at[idx2_v.at[s + 1]], bufs[1], sem)

    def drain(bufs, sem, s):
        pltpu.make_async_copy(tab_hbm.at[idx2_v.at[s]], bufs[0], sem).wait()
        pltpu.make_async_copy(tab_hbm.at[idx2_v.at[s + 1]], bufs[1], sem).wait()

    def accumulate(bufs, s):
        @pl.loop(0, BPW // LANES)
        def _(rg):
            sl = pl.ds(rg * LANES, LANES)
            rows = rg * LANES + iota
            c0 = lax.shift_left(jnp.bitwise_and(idx_v[s, sl], one), 6)
            c1 = lax.shift_left(jnp.bitwise_and(idx_v[s + 1, sl], one), 6)

            @pl.loop(0, EMBED)
            def _(d):
                v0 = plsc.load_gather(bufs[0], [rows, c0 + d])
                v1 = plsc.load_gather(bufs[1], [rows, c1 + d])
                acc_v[d, sl] = acc_v[d, sl] + (v0 + v1)

    fire((a0, a1), sem_a, 0)
    fire((b0, b1), sem_b, 2)

    @pl.loop(0, NITER)
    def _(it):
        s = it * 4
        drain((a0, a1), sem_a, s)
        accumulate((a0, a1), s)
        fire((a0, a1), sem_a, s + 4)
        drain((b0, b1), sem_b, s + 2)
        accumulate((b0, b1), s + 2)
        fire((b0, b1), sem_b, s + 6)

    last = NITER * 4
    drain((a0, a1), sem_a, last)
    accumulate((a0, a1), last)
    drain((b0, b1), sem_b, last + 2)
    accumulate((b0, b1), last + 2)

    @pl.loop(0, BPW // LANES)
    def _(g):
        sl = pl.ds(g * LANES, LANES)

        def dot_step(d, carry):
            o0, o1 = carry
            v = acc_v[d, sl]
            w0 = plsc.load_gather(wb_v, [jnp.full((LANES,), d, jnp.int32)])
            w1 = plsc.load_gather(wb_v, [jnp.full((LANES,), d + EMBED, jnp.int32)])
            return (o0 + v * w0, o1 + v * w1)

        o0, o1 = lax.fori_loop(0, EMBED, dot_step, (zeros, zeros))
        o0 = o0 + plsc.load_gather(wb_v, [jnp.full((LANES,), 2 * EMBED, jnp.int32)])
        o1 = o1 + plsc.load_gather(wb_v, [jnp.full((LANES,), 2 * EMBED + 1, jnp.int32)])
        m = jnp.maximum(o0, o1)
        e0 = jnp.exp(o0 - m)
        e1 = jnp.exp(o1 - m)
        tot = e0 + e1
        pos = (g * LANES + iota) * 2
        plsc.store_scatter(out_v, [pos], e0 / tot)
        plsc.store_scatter(out_v, [pos + 1], e1 / tot)

    pltpu.sync_copy(out_v, out_hbm.at[pl.ds(wid * 2 * BPW, 2 * BPW)])


_sc_call = pl.kernel(
    _body,
    out_type=jax.ShapeDtypeStruct((2 * BATCH,), jnp.float32),
    mesh=plsc.VectorSubcoreMesh(core_axis_name="c", subcore_axis_name="s"),
    scratch_types=[
        pltpu.VMEM((SEQ, BPW), jnp.int32),
        pltpu.VMEM((SEQ, BPW), jnp.int32),
        pltpu.VMEM((EMBED, BPW), jnp.float32),
        pltpu.VMEM((BPW, PW), jnp.float32),
        pltpu.VMEM((BPW, PW), jnp.float32),
        pltpu.VMEM((BPW, PW), jnp.float32),
        pltpu.VMEM((BPW, PW), jnp.float32),
        pltpu.VMEM((2 * BPW,), jnp.float32),
        pltpu.VMEM((136,), jnp.float32),
        pltpu.SemaphoreType.DMA,
        pltpu.SemaphoreType.DMA,
    ],
    compiler_params=pltpu.CompilerParams(
        needs_layout_passes=False, use_tc_tiling_on_sc=True),
)


@jax.jit
def kernel(input, embeddings, fc_w, fc_b):
    tab2 = _sc_convert(embeddings.T)
    scale = 1.0 / SEQ
    wb = jnp.concatenate(
        [fc_w[:, 0] * scale, fc_w[:, 1] * scale, fc_b,
         jnp.zeros((6,), jnp.float32)])
    out_flat = _sc_call(input, tab2, wb)
    return out_flat.reshape(BATCH, 2)


# speedup vs baseline: 1.4787x; 1.4787x over previous
import jax
import jax.numpy as jnp
from jax import lax
from jax.experimental import pallas as pl
from jax.experimental.pallas import tpu as pltpu
from jax.experimental.pallas import tpu_sc as plsc

SEQ = 200
BATCH = 4096
EMBED = 64
VOCAB = 1000000
PW = 2 * EMBED
NC = 2
NS = 16
NW = NC * NS
BPW = BATCH // NW
LANES = 16
CH = EMBED // LANES
NITER = SEQ // 4 - 1

NBLK = 3906
KMAIN = 122


def _conv_body(embt_hbm, tab_hbm, in0, in1, ot0, ot1, tin, tout,
               sem_i, sem_o):
    wid = lax.axis_index("s") * NC + lax.axis_index("c")
    iota = lax.iota(jnp.int32, LANES)
    rowsets = [dc * LANES + iota for dc in range(CH)]

    def fire_in(buf, k):
        g = k * NW + wid
        pltpu.async_copy(embt_hbm.at[:, pl.ds(g * 256, 256)], buf, sem_i)

    def wait_in(buf, k):
        g = k * NW + wid
        pltpu.make_async_copy(
            embt_hbm.at[:, pl.ds(g * 256, 256)], buf, sem_i).wait()

    def fire_out(buf, k):
        g = k * NW + wid
        pltpu.async_copy(buf, tab_hbm.at[pl.ds(g * 256, 256)], sem_o)

    def wait_out(buf, k):
        g = k * NW + wid
        pltpu.make_async_copy(
            buf, tab_hbm.at[pl.ds(g * 256, 256)], sem_o).wait()

    def transpose_block(inb, outb, ncols):
        @pl.loop(0, ncols, unroll=4)
        def _(i):
            col = i + jnp.zeros((LANES,), jnp.int32)
            for dc in range(CH):
                v = plsc.load_gather(inb, [rowsets[dc], col])
                outb[i, pl.ds(dc * LANES, LANES)] = v

    fire_in(in0, 0)
    fire_in(in1, 1)

    @pl.loop(0, KMAIN // 2 - 1)
    def _(j):
        k0 = 2 * j
        wait_in(in0, k0)

        @pl.when(j > 0)
        def _():
            wait_out(ot0, k0 - 2)

        transpose_block(in0, ot0, 256)
        fire_out(ot0, k0)
        fire_in(in0, k0 + 2)
        wait_in(in1, k0 + 1)

        @pl.when(j > 0)
        def _():
            wait_out(ot1, k0 - 1)

        transpose_block(in1, ot1, 256)
        fire_out(ot1, k0 + 1)
        fire_in(in1, k0 + 3)

    wait_in(in0, KMAIN - 2)
    wait_out(ot0, KMAIN - 4)
    transpose_block(in0, ot0, 256)
    fire_out(ot0, KMAIN - 2)
    wait_in(in1, KMAIN - 1)
    wait_out(ot1, KMAIN - 3)
    transpose_block(in1, ot1, 256)
    fire_out(ot1, KMAIN - 1)

    @pl.when(wid < 2)
    def _():
        fire_in(in0, KMAIN)
        wait_in(in0, KMAIN)
        wait_out(ot0, KMAIN - 2)
        transpose_block(in0, ot0, 256)
        fire_out(ot0, KMAIN)

    @pl.when(wid == 2)
    def _():
        pltpu.sync_copy(embt_hbm.at[:, pl.ds(NBLK * 256, 64)], tin)

        @pl.loop(0, 64, unroll=4)
        def _(i):
            col = i + jnp.zeros((LANES,), jnp.int32)
            for dc in range(CH):
                v = plsc.load_gather(tin, [rowsets[dc], col])
                tout[i, pl.ds(dc * LANES, LANES)] = v

        pltpu.sync_copy(tout, tab_hbm.at[pl.ds(NBLK * 256, 64)])

    @pl.when(wid < 2)
    def _():
        wait_out(ot0, KMAIN)

    @pl.when(wid >= 2)
    def _():
        wait_out(ot0, KMAIN - 2)

    wait_out(ot1, KMAIN - 1)


_sc_convert = pl.kernel(
    _conv_body,
    out_type=jax.ShapeDtypeStruct((VOCAB, PW), jnp.float32),
    mesh=plsc.VectorSubcoreMesh(core_axis_name="c", subcore_axis_name="s"),
    scratch_types=[
        pltpu.VMEM((EMBED, 256), jnp.float32),
        pltpu.VMEM((EMBED, 256), jnp.float32),
        pltpu.VMEM((256, PW), jnp.float32),
        pltpu.VMEM((256, PW), jnp.float32),
        pltpu.VMEM((EMBED, 64), jnp.float32),
        pltpu.VMEM((64, PW), jnp.float32),
        pltpu.SemaphoreType.DMA,
        pltpu.SemaphoreType.DMA,
    ],
    compiler_params=pltpu.CompilerParams(
        needs_layout_passes=False, use_tc_tiling_on_sc=True),
)


def _body(idx_hbm, tab_hbm, wb_hbm, out_hbm,
          idx_v, acc_v, a0, a1, b0, b1, out_v, wb_v,
          sem_a, sem_b):
    wid = lax.axis_index("s") * NC + lax.axis_index("c")
    iota = lax.iota(jnp.int32, LANES)

    pltpu.sync_copy(wb_hbm, wb_v)
    pltpu.sync_copy(idx_hbm.at[:, pl.ds(wid * BPW, BPW)], idx_v)

    zeros = jnp.zeros((LANES,), jnp.float32)

    @pl.loop(0, BPW)
    def _(i):
        for c in range(CH):
            acc_v[pl.ds(i * EMBED + c * LANES, LANES)] = zeros

    def fire(bufs, sem, s):
        pltpu.async_copy(tab_hbm.at[idx_v.at[s]], bufs[0], sem)
        pltpu.async_copy(tab_hbm.at[idx_v.at[s + 1]], bufs[1], sem)

    def drain(bufs, sem, s):
        pltpu.make_async_copy(tab_hbm.at[idx_v.at[s]], bufs[0], sem).wait()
        pltpu.make_async_copy(tab_hbm.at[idx_v.at[s + 1]], bufs[1], sem).wait()

    def accumulate(bufs):
        @pl.loop(0, BPW, unroll=2)
        def _(i):
            for c in range(CH):
                sl = pl.ds(c * LANES, LANES)
                asl = pl.ds(i * EMBED + c * LANES, LANES)
                acc_v[asl] = acc_v[asl] + (bufs[0][i, sl] + bufs[1][i, sl])

    fire((a0, a1), sem_a, 0)
    fire((b0, b1), sem_b, 2)

    @pl.loop(0, NITER)
    def _(it):
        s = it * 4
        drain((a0, a1), sem_a, s)
        accumulate((a0, a1))
        fire((a0, a1), sem_a, s + 4)
        drain((b0, b1), sem_b, s + 2)
        accumulate((b0, b1))
        fire((b0, b1), sem_b, s + 6)

    last = NITER * 4
    drain((a0, a1), sem_a, last)
    accumulate((a0, a1))
    drain((b0, b1), sem_b, last + 2)
    accumulate((b0, b1))

    @pl.loop(0, BPW // LANES)
    def _(g):
        rows = (g * LANES + iota) * EMBED

        def dot_step(d, carry):
            o0, o1 = carry
            col = plsc.load_gather(acc_v, [rows + d])
            w0 = plsc.load_gather(wb_v, [jnp.full((LANES,), d, jnp.int32)])
            w1 = plsc.load_gather(wb_v, [jnp.full((LANES,), d + EMBED, jnp.int32)])
            return (o0 + col * w0, o1 + col * w1)

        o0, o1 = lax.fori_loop(0, EMBED, dot_step, (zeros, zeros))
        o0 = o0 + plsc.load_gather(wb_v, [jnp.full((LANES,), 2 * EMBED, jnp.int32)])
        o1 = o1 + plsc.load_gather(wb_v, [jnp.full((LANES,), 2 * EMBED + 1, jnp.int32)])
        m = jnp.maximum(o0, o1)
        e0 = jnp.exp(o0 - m)
        e1 = jnp.exp(o1 - m)
        tot = e0 + e1
        pos = (g * LANES + iota) * 2
        plsc.store_scatter(out_v, [pos], e0 / tot)
        plsc.store_scatter(out_v, [pos + 1], e1 / tot)

    pltpu.sync_copy(out_v, out_hbm.at[pl.ds(wid * 2 * BPW, 2 * BPW)])


_sc_call = pl.kernel(
    _body,
    out_type=jax.ShapeDtypeStruct((2 * BATCH,), jnp.float32),
    mesh=plsc.VectorSubcoreMesh(core_axis_name="c", subcore_axis_name="s"),
    scratch_types=[
        pltpu.VMEM((SEQ, BPW), jnp.int32),
        pltpu.VMEM((BPW * EMBED,), jnp.float32),
        pltpu.VMEM((BPW, PW), jnp.float32),
        pltpu.VMEM((BPW, PW), jnp.float32),
        pltpu.VMEM((BPW, PW), jnp.float32),
        pltpu.VMEM((BPW, PW), jnp.float32),
        pltpu.VMEM((2 * BPW,), jnp.float32),
        pltpu.VMEM((136,), jnp.float32),
        pltpu.SemaphoreType.DMA,
        pltpu.SemaphoreType.DMA,
    ],
    compiler_params=pltpu.CompilerParams(
        needs_layout_passes=False, use_tc_tiling_on_sc=True),
)


@jax.jit
def kernel(input, embeddings, fc_w, fc_b):
    tab = _sc_convert(embeddings.T)
    scale = 1.0 / SEQ
    wb = jnp.concatenate(
        [fc_w[:, 0] * scale, fc_w[:, 1] * scale, fc_b,
         jnp.zeros((6,), jnp.float32)])
    out_flat = _sc_call(input, tab, wb)
    return out_flat.reshape(BATCH, 2)


# speedup vs baseline: 3.8325x; 2.5919x over previous
import jax
import jax.numpy as jnp
from jax import lax
from jax.experimental import pallas as pl
from jax.experimental.pallas import tpu as pltpu
from jax.experimental.pallas import tpu_sc as plsc

SEQ = 200
BATCH = 4096
EMBED = 64
NC = 2
NS = 16
NW = NC * NS
BPW = BATCH // NW
G = 4
NITER = SEQ // (2 * G)
LANES = 16
CH = EMBED // LANES


def _body(idx_hbm, table_hbm, fcw_hbm, fcb_hbm, out_hbm,
          idx_v, acc_v,
          a0, a1, a2, a3, b0, b1, b2, b3,
          out_v, fcw_v, fcb_v, sem_a, sem_b):
    wid = lax.axis_index("s") * NC + lax.axis_index("c")
    bufs_a = (a0, a1, a2, a3)
    bufs_b = (b0, b1, b2, b3)

    pltpu.sync_copy(fcw_hbm, fcw_v)
    pltpu.sync_copy(fcb_hbm, fcb_v)
    pltpu.sync_copy(idx_hbm.at[:, pl.ds(wid * BPW, BPW)], idx_v)

    def fire(bufs, sem, base):
        for k in range(G):
            pltpu.async_copy(table_hbm.at[idx_v.at[base + k]], bufs[k], sem)

    def drain(bufs, sem, base):
        for k in range(G):
            pltpu.make_async_copy(
                table_hbm.at[idx_v.at[base + k]], bufs[k], sem).wait()

    zeros = jnp.zeros((LANES,), jnp.float32)

    @pl.loop(0, BPW)
    def _(i):
        for c in range(CH):
            acc_v[pl.ds(i * EMBED + c * LANES, LANES)] = zeros

    def accumulate(bufs):
        @pl.loop(0, BPW)
        def _(i):
            for c in range(CH):
                sl = pl.ds(c * LANES, LANES)
                asl = pl.ds(i * EMBED + c * LANES, LANES)
                s = ((bufs[0][i, sl] + bufs[1][i, sl])
                     + (bufs[2][i, sl] + bufs[3][i, sl]))
                acc_v[asl] = acc_v[asl] + s

    fire(bufs_a, sem_a, 0)
    fire(bufs_b, sem_b, G)

    @pl.loop(0, NITER - 1)
    def _(it):
        base = it * (2 * G)
        drain(bufs_a, sem_a, base)
        accumulate(bufs_a)
        fire(bufs_a, sem_a, base + 2 * G)
        drain(bufs_b, sem_b, base + G)
        accumulate(bufs_b)
        fire(bufs_b, sem_b, base + 3 * G)

    last = (NITER - 1) * (2 * G)
    drain(bufs_a, sem_a, last)
    accumulate(bufs_a)
    drain(bufs_b, sem_b, last + G)
    accumulate(bufs_b)

    bias0 = fcb_v[0, :]
    bias1 = fcb_v[1, :]

    @pl.loop(0, BPW // LANES)
    def _(g):
        rows = g * LANES + lax.iota(jnp.int32, LANES)

        def dot_step(d, carry):
            o0, o1 = carry
            col = plsc.load_gather(acc_v, [rows * EMBED + d])
            return (o0 + col * fcw_v[0, d, :], o1 + col * fcw_v[1, d, :])

        o0, o1 = lax.fori_loop(0, EMBED, dot_step, (zeros, zeros))
        o0 = o0 + bias0
        o1 = o1 + bias1
        m = jnp.maximum(o0, o1)
        e0 = jnp.exp(o0 - m)
        e1 = jnp.exp(o1 - m)
        tot = e0 + e1
        pos = (g * LANES + lax.iota(jnp.int32, LANES)) * 2
        plsc.store_scatter(out_v, [pos], e0 / tot)
        plsc.store_scatter(out_v, [pos + 1], e1 / tot)

    pltpu.sync_copy(out_v, out_hbm.at[pl.ds(wid * 2 * BPW, 2 * BPW)])


_sc_call = pl.kernel(
    _body,
    out_type=jax.ShapeDtypeStruct((2 * BATCH,), jnp.float32),
    mesh=plsc.VectorSubcoreMesh(core_axis_name="c", subcore_axis_name="s"),
    scratch_types=[
        pltpu.VMEM((SEQ, BPW), jnp.int32),
        pltpu.VMEM((BPW * EMBED,), jnp.float32),
    ] + [pltpu.VMEM((BPW, EMBED), jnp.float32) for _ in range(2 * G)]
    + [
        pltpu.VMEM((2 * BPW,), jnp.float32),
        pltpu.VMEM((2, EMBED, LANES), jnp.float32),
        pltpu.VMEM((2, LANES), jnp.float32),
        pltpu.SemaphoreType.DMA,
        pltpu.SemaphoreType.DMA,
    ],
    compiler_params=pltpu.CompilerParams(
        needs_layout_passes=False, use_tc_tiling_on_sc=False),
)


@jax.jit
def kernel(input, embeddings, fc_w, fc_b):
    fcw_t = jnp.broadcast_to(
        (fc_w.T * (1.0 / SEQ))[:, :, None], (2, EMBED, LANES))
    fcb_p = jnp.broadcast_to(fc_b[:, None], (2, LANES))
    out_flat = _sc_call(input, embeddings, fcw_t, fcb_p)
    return out_flat.reshape(BATCH, 2)
